# baseline (speedup 1.0000x reference)
import functools

import jax
import jax.numpy as jnp
from jax import lax
from jax.experimental import pallas as pl
from jax.experimental.pallas import tpu as pltpu
from jax.experimental.pallas import tpu_sc as plsc

_B, _S, _D = 4, 4096, 2048
_NC, _NS = 2, 16
_NW = _NC * _NS
_ROWS = _B * _S
_RPW = _ROWS // _NW
_NBUF = 4
_SCH = 8
_DCH = 4
_SROWS = 256
_DROWS = _RPW - _SROWS
_NSG = _SROWS // (_SCH * _NBUF)
_NDG = _DROWS // (_DCH * _NBUF)
_LANES = 16


def _gather_body(x_hbm, perm_hbm, out_hbm, idx_v, buf_v, sp_idx, idx_sm,
                 sp, *sems):
    sem_g = sems[0:_NBUF]
    sem_s = sems[_NBUF:2 * _NBUF]
    sem_in = sems[2 * _NBUF:3 * _NBUF]
    sem_out = sems[3 * _NBUF:4 * _NBUF]
    cid = lax.axis_index("c")
    sid = lax.axis_index("s")
    wid = sid * _NC + cid
    base = wid * _RPW
    b = base // _S
    i0 = base - b * _S
    off = b * _S

    pltpu.sync_copy(perm_hbm.at[pl.ds(i0, _RPW)], sp_idx.at[sid])
    pltpu.sync_copy(sp_idx.at[sid], idx_sm)
    pltpu.sync_copy(perm_hbm.at[pl.ds(i0, _SROWS)], idx_v)
    off_vec = jnp.full((_LANES,), off, dtype=jnp.int32)
    for j in range(_SROWS // _LANES):
        sl = pl.ds(j * _LANES, _LANES)
        idx_v[sl] = idx_v[sl] + off_vec

    def g_copy(g, slot):
        idx_slice = idx_v.at[pl.ds(g * _SCH, _SCH)]
        return pltpu.make_async_copy(
            x_hbm.at[idx_slice], buf_v.at[slot], sem_g[slot])

    def s_copy(g, slot):
        return pltpu.make_async_copy(
            buf_v.at[slot], out_hbm.at[pl.ds(base + g * _SCH, _SCH)],
            sem_s[slot])

    spbase = sid * (_NBUF * _DCH)

    def fill(d, slot):
        for j in range(_DCH):
            r = idx_sm[_SROWS + d * _DCH + j] + off
            pltpu.make_async_copy(
                x_hbm.at[pl.ds(r, 1)],
                sp.at[pl.ds(spbase + slot * _DCH + j, 1)],
                sem_in[slot]).start()

    def drain_fill(slot):
        for j in range(_DCH):
            pltpu.make_async_copy(
                x_hbm.at[pl.ds(0, 1)],
                sp.at[pl.ds(spbase + slot * _DCH + j, 1)],
                sem_in[slot]).wait()

    def out_copy(d, slot):
        return pltpu.make_async_copy(
            sp.at[pl.ds(spbase + slot * _DCH, _DCH)],
            out_hbm.at[pl.ds(base + _SROWS + d * _DCH, _DCH)],
            sem_out[slot])

    def dma_consume_prefetch(d0, last):
        for s in range(_NBUF):
            drain_fill(s)
            out_copy(d0 + s, s).start()
        for s in range(_NBUF):
            out_copy(d0 + s, s).wait()
            if not last:
                fill(d0 + _NBUF + s, s)

    for s in range(_NBUF):
        g_copy(s, s).start()
        fill(s, s)

    def group(i, carry):
        g0s = i * _NBUF
        d0 = 2 * i * _NBUF
        for s in range(_NBUF):
            g_copy(g0s + s, s).wait()
            s_copy(g0s + s, s).start()
        dma_consume_prefetch(d0, False)
        for s in range(_NBUF):
            s_copy(g0s + s, s).wait()
            g_copy(g0s + _NBUF + s, s).start()
        dma_consume_prefetch(d0 + _NBUF, False)
        return carry

    lax.fori_loop(0, _NSG - 1, group, 0)

    gls = (_NSG - 1) * _NBUF
    dl = (_NDG - 2) * _NBUF
    for s in range(_NBUF):
        g_copy(gls + s, s).wait()
        s_copy(gls + s, s).start()
    dma_consume_prefetch(dl, False)
    dma_consume_prefetch(dl + _NBUF, True)
    for s in range(_NBUF):
        s_copy(gls + s, s).wait()


@jax.jit
def kernel(x, perm):
    x2 = x.reshape(_ROWS, _D)
    p32 = perm.astype(jnp.int32)
    mesh = plsc.VectorSubcoreMesh(core_axis_name="c", subcore_axis_name="s")
    run = pl.kernel(
        _gather_body,
        mesh=mesh,
        out_type=jax.ShapeDtypeStruct((_ROWS, _D), jnp.float32),
        scratch_types=[
            pltpu.VMEM((_SROWS,), jnp.int32),
            pltpu.VMEM((_NBUF, _SCH, _D), jnp.float32),
            pltpu.VMEM_SHARED((_NS, _RPW), jnp.int32),
            pltpu.SMEM((_RPW,), jnp.int32),
            pltpu.VMEM_SHARED((_NS * _NBUF * _DCH, _D), jnp.float32),
        ] + [pltpu.SemaphoreType.DMA] * (4 * _NBUF),
    )
    out = run(x2, p32)
    return out.reshape(_B, _S, _D)


# speedup vs baseline: 1.0294x; 1.0294x over previous
import functools

import jax
import jax.numpy as jnp
from jax import lax
from jax.experimental import pallas as pl
from jax.experimental.pallas import tpu as pltpu
from jax.experimental.pallas import tpu_sc as plsc

_B, _S, _D = 4, 4096, 2048
_NC, _NS = 2, 16
_NW = _NC * _NS
_ROWS = _B * _S
_RPW = _ROWS // _NW
_CHUNK = 8
_NBUF = 6
_LOOK = 3
_NCHUNK = _RPW // _CHUNK


def _gather_body(x_hbm, perm_hbm, out_hbm, sp_idx, idx_sm, sp, *sems):
    sem_in = sems[:_NBUF]
    sem_out = sems[_NBUF:]
    cid = lax.axis_index("c")
    sid = lax.axis_index("s")
    wid = sid * _NC + cid
    base = wid * _RPW
    b = base // _S
    i0 = base - b * _S
    off = b * _S

    pltpu.sync_copy(perm_hbm.at[pl.ds(i0, _RPW)], sp_idx.at[sid])
    pltpu.sync_copy(sp_idx.at[sid], idx_sm)

    spbase = sid * (_NBUF * _CHUNK)

    def fill(g, slot):
        for j in range(_CHUNK):
            r = idx_sm[g * _CHUNK + j] + off
            pltpu.make_async_copy(
                x_hbm.at[pl.ds(r, 1)],
                sp.at[pl.ds(spbase + slot * _CHUNK + j, 1)],
                sem_in[slot]).start()

    def drain_fill(slot):
        for j in range(_CHUNK):
            pltpu.make_async_copy(
                x_hbm.at[pl.ds(0, 1)],
                sp.at[pl.ds(spbase + slot * _CHUNK + j, 1)],
                sem_in[slot]).wait()

    def out_copy(g, slot):
        return pltpu.make_async_copy(
            sp.at[pl.ds(spbase + slot * _CHUNK, _CHUNK)],
            out_hbm.at[pl.ds(base + g * _CHUNK, _CHUNK)],
            sem_out[slot])

    def position(g):
        pf = g + _LOOK
        if pf < _NCHUNK:
            if pf - _NBUF >= 0:
                out_copy(pf - _NBUF, pf % _NBUF).wait()
            fill(pf, pf % _NBUF)
        drain_fill(g % _NBUF)
        out_copy(g, g % _NBUF).start()

    for g in range(_LOOK):
        fill(g, g)
    for g in range(7):
        position(g)

    def steady(t, carry):
        for bb in range(_NBUF):
            g = 7 + t * _NBUF + bb
            slot_c = (7 + bb) % _NBUF
            slot_p = (7 + bb + _LOOK) % _NBUF
            out_copy(g + _LOOK - _NBUF, slot_p).wait()
            fill(g + _LOOK, slot_p)
            drain_fill(slot_c)
            out_copy(g, slot_c).start()
        return carry

    lax.fori_loop(0, 9, steady, 0)

    for g in range(61, _NCHUNK):
        position(g)
    for g in range(_NCHUNK - _NBUF, _NCHUNK):
        out_copy(g, g % _NBUF).wait()


@jax.jit
def kernel(x, perm):
    x2 = x.reshape(_ROWS, _D)
    p32 = perm.astype(jnp.int32)
    mesh = plsc.VectorSubcoreMesh(core_axis_name="c", subcore_axis_name="s")
    run = pl.kernel(
        _gather_body,
        mesh=mesh,
        out_type=jax.ShapeDtypeStruct((_ROWS, _D), jnp.float32),
        scratch_types=[
            pltpu.VMEM_SHARED((_NS, _RPW), jnp.int32),
            pltpu.SMEM((_RPW,), jnp.int32),
            pltpu.VMEM_SHARED((_NS * _NBUF * _CHUNK, _D), jnp.float32),
        ] + [pltpu.SemaphoreType.DMA] * (2 * _NBUF),
    )
    out = run(x2, p32)
    return out.reshape(_B, _S, _D)


# speedup vs baseline: 1.0312x; 1.0018x over previous
import functools

import jax
import jax.numpy as jnp
from jax import lax
from jax.experimental import pallas as pl
from jax.experimental.pallas import tpu as pltpu
from jax.experimental.pallas import tpu_sc as plsc

_B, _S, _D = 4, 4096, 2048
_NC, _NS = 2, 16
_NW = _NC * _NS
_ROWS = _B * _S
_RPW = _ROWS // _NW
_CHUNK = 8
_NBUF = 4
_NCHUNK = _RPW // _CHUNK
_LANES = 16


def _gather_body(x_hbm, perm_hbm, out_hbm, sp_idx, idx_sm, sp, *sems):
    sem_in = sems[:_NBUF]
    sem_out = sems[_NBUF:]
    cid = lax.axis_index("c")
    sid = lax.axis_index("s")
    wid = sid * _NC + cid
    base = wid * _RPW
    b = base // _S
    i0 = base - b * _S
    off = b * _S

    pltpu.sync_copy(perm_hbm.at[pl.ds(i0, _RPW)], sp_idx.at[sid])
    pltpu.sync_copy(sp_idx.at[sid], idx_sm)

    spbase = sid * (_NBUF * _CHUNK)

    def fill(g, slot):
        for j in range(_CHUNK):
            r = idx_sm[g * _CHUNK + j] + off
            pltpu.make_async_copy(
                x_hbm.at[pl.ds(r, 1)],
                sp.at[pl.ds(spbase + slot * _CHUNK + j, 1)],
                sem_in[slot]).start()

    def drain_fill(g, slot):
        for j in range(_CHUNK):
            pltpu.make_async_copy(
                x_hbm.at[pl.ds(0, 1)],
                sp.at[pl.ds(spbase + slot * _CHUNK + j, 1)],
                sem_in[slot]).wait()

    def out_copy(g, slot):
        return pltpu.make_async_copy(
            sp.at[pl.ds(spbase + slot * _CHUNK, _CHUNK)],
            out_hbm.at[pl.ds(base + g * _CHUNK, _CHUNK)],
            sem_out[slot])

    for s in range(_NBUF):
        fill(s, s)

    def group(i, carry):
        g0 = i * _NBUF
        for s in range(_NBUF):
            drain_fill(g0 + s, s)
            out_copy(g0 + s, s).start()
        for s in range(_NBUF):
            out_copy(g0 + s, s).wait()
            fill(g0 + _NBUF + s, s)
        return carry

    lax.fori_loop(0, _NCHUNK // _NBUF - 1, group, 0)

    gl = (_NCHUNK // _NBUF - 1) * _NBUF
    for s in range(_NBUF):
        drain_fill(gl + s, s)
        out_copy(gl + s, s).start()
    for s in range(_NBUF):
        out_copy(gl + s, s).wait()


@jax.jit
def kernel(x, perm):
    x2 = x.reshape(_ROWS, _D)
    p32 = perm.astype(jnp.int32)
    mesh = plsc.VectorSubcoreMesh(core_axis_name="c", subcore_axis_name="s")
    run = pl.kernel(
        _gather_body,
        mesh=mesh,
        out_type=jax.ShapeDtypeStruct((_ROWS, _D), jnp.float32),
        scratch_types=[
            pltpu.VMEM_SHARED((_NS, _RPW), jnp.int32),
            pltpu.SMEM((_RPW,), jnp.int32),
            pltpu.VMEM_SHARED((_NS * _NBUF * _CHUNK, _D), jnp.float32),
        ] + [pltpu.SemaphoreType.DMA] * (2 * _NBUF),
    )
    out = run(x2, p32)
    return out.reshape(_B, _S, _D)
